# baseline (speedup 1.0000x reference)
import functools

import jax
import jax.numpy as jnp
from jax import lax
from jax.experimental import pallas as pl
from jax.experimental.pallas import tpu as pltpu
from jax.experimental.pallas import tpu_sc as plsc

D = 1024
E = 8
K = 2
T = 2048
DFF = 4 * D

M = 512
F = 512
FC = DFF // F
NT = (T * K) // M + E
R = NT * M

TB = 256

NC = 2
NS = 16
NW = NC * NS
LANES = 16

def _sc_mesh():
    return plsc.VectorSubcoreMesh(
        core_axis_name="c", subcore_axis_name="s",
        num_cores=NC, num_subcores=NS)


def _router_body(x_ref, g_ref, b_ref, wr_ref, xn_ref, p_ref):
    x = x_ref[...]
    mu = jnp.mean(x, axis=1, keepdims=True)
    var = jnp.mean((x - mu) ** 2, axis=1, keepdims=True)
    xn = (x - mu) * lax.rsqrt(var + 1e-5) * g_ref[...] + b_ref[...]
    xn_ref[...] = xn
    logits = lax.dot_general(
        xn, wr_ref[...], (((1,), (1,)), ((), ())),
        preferred_element_type=jnp.float32,
    )
    z = logits * (1.0 / 1.5)
    z = z - jnp.max(z, axis=1, keepdims=True)
    ez = jnp.exp(z)
    p_ref[...] = ez / jnp.sum(ez, axis=1, keepdims=True)


def _router(x2d, ln_g, ln_b, Wr):
    return pl.pallas_call(
        _router_body,
        grid=(T // TB,),
        in_specs=[
            pl.BlockSpec((TB, D), lambda t: (t, 0)),
            pl.BlockSpec((1, D), lambda t: (0, 0)),
            pl.BlockSpec((1, D), lambda t: (0, 0)),
            pl.BlockSpec((E, D), lambda t: (0, 0)),
        ],
        out_specs=[
            pl.BlockSpec((TB, D), lambda t: (t, 0)),
            pl.BlockSpec((TB, E), lambda t: (t, 0)),
        ],
        out_shape=[
            jax.ShapeDtypeStruct((T, D), jnp.float32),
            jax.ShapeDtypeStruct((T, E), jnp.float32),
        ],
    )(x2d, ln_g.reshape(1, D), ln_b.reshape(1, D), Wr)


def _dispatch_metadata(probs):
    w, idx = lax.top_k(probs, K)
    flat_e = idx.reshape(-1)
    order = jnp.argsort(flat_e, stable=True)
    sorted_e = flat_e[order]

    counts = jnp.sum(flat_e[None, :] == jnp.arange(E)[:, None], axis=1)
    tiles_e = (counts + M - 1) // M
    tile_start = jnp.concatenate([jnp.zeros((1,), jnp.int32),
                                  jnp.cumsum(tiles_e).astype(jnp.int32)])
    pad_start = tile_start[:E] * M
    sort_start = jnp.concatenate([jnp.zeros((1,), jnp.int32),
                                  jnp.cumsum(counts).astype(jnp.int32)])[:E]

    slot_pos = jnp.arange(T * K, dtype=jnp.int32)
    row_of_slot = pad_start[sorted_e] + slot_pos - sort_start[sorted_e]

    token_of_row = jnp.zeros((R,), jnp.int32).at[row_of_slot].set(
        (order // K).astype(jnp.int32), mode="promise_in_bounds")
    w_of_row = jnp.zeros((R,), jnp.float32).at[row_of_slot].set(
        w.reshape(-1)[order], mode="promise_in_bounds")
    row_of_flat = jnp.zeros((T * K,), jnp.int32).at[order].set(
        row_of_slot.astype(jnp.int32), mode="promise_in_bounds")
    r01 = row_of_flat.reshape(T, K)

    n_act = tile_start[E]
    raw_tile_e = jnp.searchsorted(
        tile_start[1:], jnp.arange(NT, dtype=jnp.int32), side="right"
    ).astype(jnp.int32)
    active = (jnp.arange(NT, dtype=jnp.int32) < n_act).astype(jnp.int32)
    last_e = raw_tile_e[jnp.maximum(n_act - 1, 0)]
    tile_e = jnp.where(active == 1, jnp.minimum(raw_tile_e, E - 1), last_e)
    blk = jnp.where(active == 1, jnp.arange(NT, dtype=jnp.int32),
                    jnp.maximum(n_act - 1, 0)).astype(jnp.int32)
    return (w_of_row, token_of_row, r01[:, 0], r01[:, 1], tile_e, active, blk)


GCH = 64


def _gather_body(xn_hbm, tok_hbm, out_hbm, idx_v, rows_v, sem):
    wid = lax.axis_index("s") * NC + lax.axis_index("c")
    per_w = R // NW

    def chunk(i, _):
        base = wid * per_w + i * GCH
        pltpu.sync_copy(tok_hbm.at[pl.ds(base, GCH)], idx_v)
        pltpu.async_copy(xn_hbm.at[idx_v], rows_v, sem).wait()
        pltpu.sync_copy(rows_v, out_hbm.at[pl.ds(base, GCH)])
        return _

    lax.fori_loop(0, per_w // GCH, chunk, None)


def _sc_gather(xn, token_of_row):
    k = functools.partial(
        pl.kernel,
        out_type=jax.ShapeDtypeStruct((R, D), jnp.float32),
        mesh=_sc_mesh(),
        scratch_types=[
            pltpu.VMEM((GCH,), jnp.int32),
            pltpu.VMEM((GCH, D), jnp.float32),
            pltpu.SemaphoreType.DMA,
        ],
    )(_gather_body)
    return k(xn, token_of_row)


def _mm_body(te_ref, act_ref, blk_ref, xs_ref, w1_ref, w2_ref, wr_ref,
             ys_ref):
    t = pl.program_id(0)
    f = pl.program_id(1)
    is_act = act_ref[t] == 1

    @pl.when(jnp.logical_and(is_act, f == 0))
    def _():
        ys_ref[...] = jnp.zeros_like(ys_ref)

    @pl.when(is_act)
    def _():
        x = xs_ref[...].astype(jnp.bfloat16)
        w1 = w1_ref[0].astype(jnp.bfloat16)
        h = lax.dot_general(x, w1, (((1,), (1,)), ((), ())),
                            preferred_element_type=jnp.float32)
        h = h * lax.logistic(h)
        w2 = w2_ref[0].astype(jnp.bfloat16)
        y = lax.dot_general(h.astype(jnp.bfloat16), w2,
                            (((1,), (1,)), ((), ())),
                            preferred_element_type=jnp.float32)
        ys_ref[...] += y

    @pl.when(jnp.logical_and(is_act, f == FC - 1))
    def _():
        ys_ref[...] *= wr_ref[0]


def _grouped_ffn(xs, W1, W2, w_of_row, tile_e, active, blk):
    wrow = w_of_row.reshape(NT, M, 1)
    grid_spec = pltpu.PrefetchScalarGridSpec(
        num_scalar_prefetch=3,
        grid=(NT, FC),
        in_specs=[
            pl.BlockSpec((M, D), lambda t, f, te, act, blk: (blk[t], 0)),
            pl.BlockSpec(
                (1, F, D),
                lambda t, f, te, act, blk:
                    (te[t], jnp.where(act[t] == 1, f, FC - 1), 0)),
            pl.BlockSpec(
                (1, D, F),
                lambda t, f, te, act, blk:
                    (te[t], 0, jnp.where(act[t] == 1, f, FC - 1))),
            pl.BlockSpec((1, M, 1), lambda t, f, te, act, blk: (blk[t], 0, 0)),
        ],
        out_specs=pl.BlockSpec((M, D), lambda t, f, te, act, blk: (blk[t], 0)),
    )
    return pl.pallas_call(
        _mm_body,
        grid_spec=grid_spec,
        out_shape=jax.ShapeDtypeStruct((R, D), jnp.float32),
        compiler_params=pltpu.CompilerParams(
            dimension_semantics=("arbitrary", "arbitrary")),
    )(tile_e, active, blk, xs, W1, W2, wrow)


CCH = 16


def _combine_body(ys_hbm, r0_hbm, r1_hbm, x_hbm, out_hbm,
                  i0_v, i1_v, a_v, b_v, c_v, sem0, sem1):
    wid = lax.axis_index("s") * NC + lax.axis_index("c")
    per_w = T // NW

    def chunk(i, _):
        base = wid * per_w + i * CCH
        pltpu.sync_copy(r0_hbm.at[pl.ds(base, CCH)], i0_v)
        pltpu.sync_copy(r1_hbm.at[pl.ds(base, CCH)], i1_v)
        cp0 = pltpu.async_copy(ys_hbm.at[i0_v], a_v, sem0)
        cp1 = pltpu.async_copy(ys_hbm.at[i1_v], b_v, sem1)
        pltpu.sync_copy(x_hbm.at[pl.ds(base, CCH)], c_v)
        cp0.wait()
        cp1.wait()

        def row(r, _):
            for j in range(D // LANES):
                s = pl.ds(j * LANES, LANES)
                c_v[r, s] = c_v[r, s] + a_v[r, s] + b_v[r, s]
            return _

        lax.fori_loop(0, CCH, row, None)
        pltpu.sync_copy(c_v, out_hbm.at[pl.ds(base, CCH)])
        return _

    lax.fori_loop(0, per_w // CCH, chunk, None)


def _sc_combine(ys, r0, r1, x2d):
    k = functools.partial(
        pl.kernel,
        out_type=jax.ShapeDtypeStruct((T, D), jnp.float32),
        mesh=_sc_mesh(),
        scratch_types=[
            pltpu.VMEM((CCH,), jnp.int32),
            pltpu.VMEM((CCH,), jnp.int32),
            pltpu.VMEM((CCH, D), jnp.float32),
            pltpu.VMEM((CCH, D), jnp.float32),
            pltpu.VMEM((CCH, D), jnp.float32),
            pltpu.SemaphoreType.DMA,
            pltpu.SemaphoreType.DMA,
        ],
    )(_combine_body)
    return k(ys, r0, r1, x2d)


def kernel(x, ln_g, ln_b, Wr, W1, W2):
    x2d = x.reshape(T, D)
    xn, probs = _router(x2d, ln_g, ln_b, Wr)
    w_of_row, token_of_row, r0, r1, tile_e, active, blk = _dispatch_metadata(
        probs)
    xs = _sc_gather(xn, token_of_row)
    ys = _grouped_ffn(xs, W1, W2, w_of_row, tile_e, active, blk)
    out = _sc_combine(ys, r0, r1, x2d)
    return out.reshape(1, T, D)


# speedup vs baseline: 1.4440x; 1.4440x over previous
import functools

import jax
import jax.numpy as jnp
from jax import lax
from jax.experimental import pallas as pl
from jax.experimental.pallas import tpu as pltpu
from jax.experimental.pallas import tpu_sc as plsc

D = 1024
E = 8
K = 2
T = 2048
DFF = 4 * D

M = 512
F = 512
FC = DFF // F
NT = (T * K) // M + E
R = NT * M

TB = 256

NC = 2
NS = 16
NW = NC * NS
LANES = 16

def _sc_mesh():
    return plsc.VectorSubcoreMesh(
        core_axis_name="c", subcore_axis_name="s",
        num_cores=NC, num_subcores=NS)


def _router_body(x_ref, g_ref, b_ref, wr_ref, xn_ref, p_ref):
    x = x_ref[...]
    mu = jnp.mean(x, axis=1, keepdims=True)
    var = jnp.mean((x - mu) ** 2, axis=1, keepdims=True)
    xn = (x - mu) * lax.rsqrt(var + 1e-5) * g_ref[...] + b_ref[...]
    xn_ref[...] = xn.astype(jnp.bfloat16)
    logits = lax.dot_general(
        xn, wr_ref[...], (((1,), (1,)), ((), ())),
        preferred_element_type=jnp.float32,
    )
    z = logits * (1.0 / 1.5)
    z = z - jnp.max(z, axis=1, keepdims=True)
    ez = jnp.exp(z)
    p_ref[...] = ez / jnp.sum(ez, axis=1, keepdims=True)


def _router(x2d, ln_g, ln_b, Wr):
    return pl.pallas_call(
        _router_body,
        grid=(T // TB,),
        in_specs=[
            pl.BlockSpec((TB, D), lambda t: (t, 0)),
            pl.BlockSpec((1, D), lambda t: (0, 0)),
            pl.BlockSpec((1, D), lambda t: (0, 0)),
            pl.BlockSpec((E, D), lambda t: (0, 0)),
        ],
        out_specs=[
            pl.BlockSpec((TB, D), lambda t: (t, 0)),
            pl.BlockSpec((TB, E), lambda t: (t, 0)),
        ],
        out_shape=[
            jax.ShapeDtypeStruct((T, D), jnp.bfloat16),
            jax.ShapeDtypeStruct((T, E), jnp.float32),
        ],
    )(x2d, ln_g.reshape(1, D), ln_b.reshape(1, D), Wr)


def _dispatch_metadata(probs):
    w, idx = lax.top_k(probs, K)
    flat_e = idx.reshape(-1)
    order = jnp.argsort(flat_e, stable=True)
    sorted_e = flat_e[order]

    counts = jnp.sum(flat_e[None, :] == jnp.arange(E)[:, None], axis=1)
    tiles_e = (counts + M - 1) // M
    tile_start = jnp.concatenate([jnp.zeros((1,), jnp.int32),
                                  jnp.cumsum(tiles_e).astype(jnp.int32)])
    pad_start = tile_start[:E] * M
    sort_start = jnp.concatenate([jnp.zeros((1,), jnp.int32),
                                  jnp.cumsum(counts).astype(jnp.int32)])[:E]

    slot_pos = jnp.arange(T * K, dtype=jnp.int32)
    row_of_slot = pad_start[sorted_e] + slot_pos - sort_start[sorted_e]

    token_of_row = jnp.zeros((R,), jnp.int32).at[row_of_slot].set(
        (order // K).astype(jnp.int32), mode="promise_in_bounds")
    w_of_row = jnp.zeros((R,), jnp.float32).at[row_of_slot].set(
        w.reshape(-1)[order], mode="promise_in_bounds")
    row_of_flat = jnp.zeros((T * K,), jnp.int32).at[order].set(
        row_of_slot.astype(jnp.int32), mode="promise_in_bounds")
    r01 = row_of_flat.reshape(T, K)

    n_act = tile_start[E]
    raw_tile_e = jnp.searchsorted(
        tile_start[1:], jnp.arange(NT, dtype=jnp.int32), side="right"
    ).astype(jnp.int32)
    active = (jnp.arange(NT, dtype=jnp.int32) < n_act).astype(jnp.int32)
    last_e = raw_tile_e[jnp.maximum(n_act - 1, 0)]
    tile_e = jnp.where(active == 1, jnp.minimum(raw_tile_e, E - 1), last_e)
    blk = jnp.where(active == 1, jnp.arange(NT, dtype=jnp.int32),
                    jnp.maximum(n_act - 1, 0)).astype(jnp.int32)
    return (w_of_row, token_of_row, r01[:, 0], r01[:, 1], tile_e, active, blk)


def _mm_body(te_ref, act_ref, blk_ref, xnb_ref, tok_ref, w1_ref, w2_ref,
             wr_ref, ys_ref, xg_ref):
    t = pl.program_id(0)
    f = pl.program_id(1)
    is_act = act_ref[t] == 1

    @pl.when(jnp.logical_and(is_act, f == 0))
    def _():
        ys_ref[...] = jnp.zeros_like(ys_ref)
        tok = tok_ref[0]
        onehot = (tok == lax.broadcasted_iota(jnp.int32, (M, T), 1)
                  ).astype(jnp.bfloat16)
        xg = lax.dot_general(onehot, xnb_ref[...], (((1,), (0,)), ((), ())),
                             preferred_element_type=jnp.float32)
        xg_ref[...] = xg.astype(jnp.bfloat16)

    @pl.when(is_act)
    def _():
        x = xg_ref[...]
        w1 = w1_ref[0].astype(jnp.bfloat16)
        h = lax.dot_general(x, w1, (((1,), (1,)), ((), ())),
                            preferred_element_type=jnp.float32)
        h = h * lax.logistic(h)
        w2 = w2_ref[0].astype(jnp.bfloat16)
        y = lax.dot_general(h.astype(jnp.bfloat16), w2,
                            (((1,), (1,)), ((), ())),
                            preferred_element_type=jnp.float32)
        ys_ref[...] += y

    @pl.when(jnp.logical_and(is_act, f == FC - 1))
    def _():
        ys_ref[...] *= wr_ref[0]


def _grouped_ffn(xnb, token_of_row, W1, W2, w_of_row, tile_e, active, blk):
    wrow = w_of_row.reshape(NT, M, 1)
    tok3 = token_of_row.reshape(NT, M, 1)
    grid_spec = pltpu.PrefetchScalarGridSpec(
        num_scalar_prefetch=3,
        grid=(NT, FC),
        in_specs=[
            pl.BlockSpec((T, D), lambda t, f, te, act, blk: (0, 0)),
            pl.BlockSpec((1, M, 1), lambda t, f, te, act, blk: (blk[t], 0, 0)),
            pl.BlockSpec(
                (1, F, D),
                lambda t, f, te, act, blk:
                    (te[t], jnp.where(act[t] == 1, f, FC - 1), 0)),
            pl.BlockSpec(
                (1, D, F),
                lambda t, f, te, act, blk:
                    (te[t], 0, jnp.where(act[t] == 1, f, FC - 1))),
            pl.BlockSpec((1, M, 1), lambda t, f, te, act, blk: (blk[t], 0, 0)),
        ],
        out_specs=pl.BlockSpec((M, D), lambda t, f, te, act, blk: (blk[t], 0)),
        scratch_shapes=[pltpu.VMEM((M, D), jnp.bfloat16)],
    )
    return pl.pallas_call(
        _mm_body,
        grid_spec=grid_spec,
        out_shape=jax.ShapeDtypeStruct((R, D), jnp.float32),
        compiler_params=pltpu.CompilerParams(
            dimension_semantics=("arbitrary", "arbitrary")),
    )(tile_e, active, blk, xnb, tok3, W1, W2, wrow)


CCH = 16


def _combine_body(ys_hbm, r0_hbm, r1_hbm, x_hbm, out_hbm,
                  i0_v, i1_v, a_v, b_v, c_v, sem0, sem1):
    wid = lax.axis_index("s") * NC + lax.axis_index("c")
    per_w = T // NW

    def chunk(i, _):
        base = wid * per_w + i * CCH
        pltpu.sync_copy(r0_hbm.at[pl.ds(base, CCH)], i0_v)
        pltpu.sync_copy(r1_hbm.at[pl.ds(base, CCH)], i1_v)
        cp0 = pltpu.async_copy(ys_hbm.at[i0_v], a_v, sem0)
        cp1 = pltpu.async_copy(ys_hbm.at[i1_v], b_v, sem1)
        pltpu.sync_copy(x_hbm.at[pl.ds(base, CCH)], c_v)
        cp0.wait()
        cp1.wait()

        def row(r, _):
            for j in range(D // LANES):
                s = pl.ds(j * LANES, LANES)
                c_v[r, s] = c_v[r, s] + a_v[r, s] + b_v[r, s]
            return _

        lax.fori_loop(0, CCH, row, None)
        pltpu.sync_copy(c_v, out_hbm.at[pl.ds(base, CCH)])
        return _

    lax.fori_loop(0, per_w // CCH, chunk, None)


def _sc_combine(ys, r0, r1, x2d):
    k = functools.partial(
        pl.kernel,
        out_type=jax.ShapeDtypeStruct((T, D), jnp.float32),
        mesh=_sc_mesh(),
        scratch_types=[
            pltpu.VMEM((CCH,), jnp.int32),
            pltpu.VMEM((CCH,), jnp.int32),
            pltpu.VMEM((CCH, D), jnp.float32),
            pltpu.VMEM((CCH, D), jnp.float32),
            pltpu.VMEM((CCH, D), jnp.float32),
            pltpu.SemaphoreType.DMA,
            pltpu.SemaphoreType.DMA,
        ],
    )(_combine_body)
    return k(ys, r0, r1, x2d)


def kernel(x, ln_g, ln_b, Wr, W1, W2):
    x2d = x.reshape(T, D)
    xnb, probs = _router(x2d, ln_g, ln_b, Wr)
    w_of_row, token_of_row, r0, r1, tile_e, active, blk = _dispatch_metadata(
        probs)
    ys = _grouped_ffn(xnb, token_of_row, W1, W2, w_of_row, tile_e, active,
                      blk)
    out = _sc_combine(ys, r0, r1, x2d)
    return out.reshape(1, T, D)


# speedup vs baseline: 1.6676x; 1.1548x over previous
import functools

import jax
import jax.numpy as jnp
from jax import lax
from jax.experimental import pallas as pl
from jax.experimental.pallas import tpu as pltpu
from jax.experimental.pallas import tpu_sc as plsc

D = 1024
E = 8
K = 2
T = 2048
DFF = 4 * D

M = 512
F = 512
FC = DFF // F
NT = (T * K) // M + E
R = NT * M

TB = 256

NC = 2
NS = 16
NW = NC * NS
LANES = 16

def _sc_mesh():
    return plsc.VectorSubcoreMesh(
        core_axis_name="c", subcore_axis_name="s",
        num_cores=NC, num_subcores=NS)


def _router_body(x_ref, g_ref, b_ref, wr_ref, xn_ref, p_ref):
    x = x_ref[...]
    mu = jnp.mean(x, axis=1, keepdims=True)
    var = jnp.mean((x - mu) ** 2, axis=1, keepdims=True)
    xn = (x - mu) * lax.rsqrt(var + 1e-5) * g_ref[...] + b_ref[...]
    xn_ref[...] = xn.astype(jnp.bfloat16)
    logits = lax.dot_general(
        xn, wr_ref[...], (((1,), (1,)), ((), ())),
        preferred_element_type=jnp.float32,
    )
    z = logits * (1.0 / 1.5)
    z = z - jnp.max(z, axis=1, keepdims=True)
    ez = jnp.exp(z)
    p_ref[...] = ez / jnp.sum(ez, axis=1, keepdims=True)


def _router(x2d, ln_g, ln_b, Wr):
    return pl.pallas_call(
        _router_body,
        grid=(T // TB,),
        in_specs=[
            pl.BlockSpec((TB, D), lambda t: (t, 0)),
            pl.BlockSpec((1, D), lambda t: (0, 0)),
            pl.BlockSpec((1, D), lambda t: (0, 0)),
            pl.BlockSpec((E, D), lambda t: (0, 0)),
        ],
        out_specs=[
            pl.BlockSpec((TB, D), lambda t: (t, 0)),
            pl.BlockSpec((TB, E), lambda t: (t, 0)),
        ],
        out_shape=[
            jax.ShapeDtypeStruct((T, D), jnp.bfloat16),
            jax.ShapeDtypeStruct((T, E), jnp.float32),
        ],
    )(x2d, ln_g.reshape(1, D), ln_b.reshape(1, D), Wr)


def _dispatch_metadata(probs):
    w, idx = lax.top_k(probs, K)
    flat_e = idx.reshape(-1)

    onehot = (flat_e[:, None] == jnp.arange(E)[None, :]).astype(jnp.int32)
    csum = jnp.cumsum(onehot, axis=0)
    counts = csum[-1]
    rank = jnp.take_along_axis(csum - onehot, flat_e[:, None], axis=1)[:, 0]

    tiles_e = (counts + M - 1) // M
    tile_start = jnp.concatenate([jnp.zeros((1,), jnp.int32),
                                  jnp.cumsum(tiles_e).astype(jnp.int32)])
    pad_start = tile_start[:E] * M

    row_of_flat = (pad_start[flat_e] + rank).astype(jnp.int32)
    r01 = row_of_flat.reshape(T, K)

    n_act = tile_start[E]
    raw_tile_e = jnp.searchsorted(
        tile_start[1:], jnp.arange(NT, dtype=jnp.int32), side="right"
    ).astype(jnp.int32)
    active = (jnp.arange(NT, dtype=jnp.int32) < n_act).astype(jnp.int32)
    last_e = raw_tile_e[jnp.maximum(n_act - 1, 0)]
    tile_e = jnp.where(active == 1, jnp.minimum(raw_tile_e, E - 1), last_e)
    blk = jnp.where(active == 1, jnp.arange(NT, dtype=jnp.int32),
                    jnp.maximum(n_act - 1, 0)).astype(jnp.int32)

    wh = w.astype(jnp.bfloat16)
    wl = (w - wh.astype(jnp.float32)).astype(jnp.bfloat16)
    w0cat = jnp.stack([wh[:, 0], wl[:, 0]], axis=1).reshape(1, T, 2)
    w1cat = jnp.stack([wh[:, 1], wl[:, 1]], axis=1).reshape(1, T, 2)
    return (w0cat, w1cat, r01[:, 0], r01[:, 1], tile_e, active, blk)


def _mm_body(te_ref, act_ref, blk_ref, xnb_ref, r0_ref, r1_ref, w0_ref,
             w1c_ref, wm1_ref, wm2_ref, ys_ref, xg_ref, wrow_ref):
    t = pl.program_id(0)
    f = pl.program_id(1)
    is_act = act_ref[t] == 1

    @pl.when(jnp.logical_and(is_act, f == 0))
    def _():
        ys_ref[...] = jnp.zeros_like(ys_ref)
        rid = lax.broadcasted_iota(jnp.int32, (M, T), 0) + t * M
        oh0 = (rid == r0_ref[0]).astype(jnp.bfloat16)
        oh1 = (rid == r1_ref[0]).astype(jnp.bfloat16)
        xg = lax.dot_general(oh0 + oh1, xnb_ref[...], (((1,), (0,)), ((), ())),
                             preferred_element_type=jnp.float32)
        xg_ref[...] = xg.astype(jnp.bfloat16)
        a = lax.dot_general(oh0, w0_ref[0], (((1,), (0,)), ((), ())),
                            preferred_element_type=jnp.float32)
        b = lax.dot_general(oh1, w1c_ref[0], (((1,), (0,)), ((), ())),
                            preferred_element_type=jnp.float32)
        wrow_ref[...] = jnp.sum(a + b, axis=1, keepdims=True)

    @pl.when(is_act)
    def _():
        x = xg_ref[...]
        w1 = wm1_ref[0].astype(jnp.bfloat16)
        h = lax.dot_general(x, w1, (((1,), (1,)), ((), ())),
                            preferred_element_type=jnp.float32)
        h = h * lax.logistic(h)
        w2 = wm2_ref[0].astype(jnp.bfloat16)
        y = lax.dot_general(h.astype(jnp.bfloat16), w2,
                            (((1,), (1,)), ((), ())),
                            preferred_element_type=jnp.float32)
        ys_ref[...] += y

    @pl.when(jnp.logical_and(is_act, f == FC - 1))
    def _():
        ys_ref[...] *= wrow_ref[...]


def _grouped_ffn(xnb, r0, r1, w0cat, w1cat, W1, W2, tile_e, active, blk):
    grid_spec = pltpu.PrefetchScalarGridSpec(
        num_scalar_prefetch=3,
        grid=(NT, FC),
        in_specs=[
            pl.BlockSpec((T, D), lambda t, f, te, act, blk: (0, 0)),
            pl.BlockSpec((1, 1, T), lambda t, f, te, act, blk: (0, 0, 0)),
            pl.BlockSpec((1, 1, T), lambda t, f, te, act, blk: (0, 0, 0)),
            pl.BlockSpec((1, T, 2), lambda t, f, te, act, blk: (0, 0, 0)),
            pl.BlockSpec((1, T, 2), lambda t, f, te, act, blk: (0, 0, 0)),
            pl.BlockSpec(
                (1, F, D),
                lambda t, f, te, act, blk:
                    (te[t], jnp.where(act[t] == 1, f, FC - 1), 0)),
            pl.BlockSpec(
                (1, D, F),
                lambda t, f, te, act, blk:
                    (te[t], 0, jnp.where(act[t] == 1, f, FC - 1))),
        ],
        out_specs=pl.BlockSpec((M, D), lambda t, f, te, act, blk: (blk[t], 0)),
        scratch_shapes=[pltpu.VMEM((M, D), jnp.bfloat16),
                        pltpu.VMEM((M, 1), jnp.float32)],
    )
    return pl.pallas_call(
        _mm_body,
        grid_spec=grid_spec,
        out_shape=jax.ShapeDtypeStruct((R, D), jnp.float32),
        compiler_params=pltpu.CompilerParams(
            dimension_semantics=("arbitrary", "arbitrary")),
    )(tile_e, active, blk, xnb, r0.reshape(1, 1, T), r1.reshape(1, 1, T),
      w0cat, w1cat, W1, W2)


CCH = 16


def _combine_body(ys_hbm, r0_hbm, r1_hbm, x_hbm, out_hbm,
                  i0_v, i1_v, a_v, b_v, c_v, sem0, sem1):
    wid = lax.axis_index("s") * NC + lax.axis_index("c")
    per_w = T // NW

    def chunk(i, _):
        base = wid * per_w + i * CCH
        pltpu.sync_copy(r0_hbm.at[pl.ds(base, CCH)], i0_v)
        pltpu.sync_copy(r1_hbm.at[pl.ds(base, CCH)], i1_v)
        cp0 = pltpu.async_copy(ys_hbm.at[i0_v], a_v, sem0)
        cp1 = pltpu.async_copy(ys_hbm.at[i1_v], b_v, sem1)
        pltpu.sync_copy(x_hbm.at[pl.ds(base, CCH)], c_v)
        cp0.wait()
        cp1.wait()

        def row(r, _):
            for j in range(D // LANES):
                s = pl.ds(j * LANES, LANES)
                c_v[r, s] = c_v[r, s] + a_v[r, s] + b_v[r, s]
            return _

        lax.fori_loop(0, CCH, row, None)
        pltpu.sync_copy(c_v, out_hbm.at[pl.ds(base, CCH)])
        return _

    lax.fori_loop(0, per_w // CCH, chunk, None)


def _sc_combine(ys, r0, r1, x2d):
    k = functools.partial(
        pl.kernel,
        out_type=jax.ShapeDtypeStruct((T, D), jnp.float32),
        mesh=_sc_mesh(),
        scratch_types=[
            pltpu.VMEM((CCH,), jnp.int32),
            pltpu.VMEM((CCH,), jnp.int32),
            pltpu.VMEM((CCH, D), jnp.float32),
            pltpu.VMEM((CCH, D), jnp.float32),
            pltpu.VMEM((CCH, D), jnp.float32),
            pltpu.SemaphoreType.DMA,
            pltpu.SemaphoreType.DMA,
        ],
    )(_combine_body)
    return k(ys, r0, r1, x2d)


def kernel(x, ln_g, ln_b, Wr, W1, W2):
    x2d = x.reshape(T, D)
    xnb, probs = _router(x2d, ln_g, ln_b, Wr)
    w0cat, w1cat, r0, r1, tile_e, active, blk = _dispatch_metadata(probs)
    ys = _grouped_ffn(xnb, r0, r1, w0cat, w1cat, W1, W2, tile_e, active, blk)
    out = _sc_combine(ys, r0, r1, x2d)
    return out.reshape(1, T, D)


# speedup vs baseline: 1.8968x; 1.1375x over previous
import functools

import jax
import jax.numpy as jnp
from jax import lax
from jax.experimental import pallas as pl
from jax.experimental.pallas import tpu as pltpu
from jax.experimental.pallas import tpu_sc as plsc

D = 1024
E = 8
K = 2
T = 2048
DFF = 4 * D

M = 512
F = 1024
FC = DFF // F
NT = (T * K) // M + E
R = NT * M

TB = 256

NC = 2
NS = 16
NW = NC * NS
LANES = 16

def _sc_mesh():
    return plsc.VectorSubcoreMesh(
        core_axis_name="c", subcore_axis_name="s",
        num_cores=NC, num_subcores=NS)


def _router_body(x_ref, g_ref, b_ref, wr_ref, xn_ref, p_ref):
    x = x_ref[...]
    mu = jnp.mean(x, axis=1, keepdims=True)
    var = jnp.mean((x - mu) ** 2, axis=1, keepdims=True)
    xn = (x - mu) * lax.rsqrt(var + 1e-5) * g_ref[...] + b_ref[...]
    xn_ref[...] = xn.astype(jnp.bfloat16)
    logits = lax.dot_general(
        xn, wr_ref[...], (((1,), (1,)), ((), ())),
        preferred_element_type=jnp.float32,
    )
    z = logits * (1.0 / 1.5)
    z = z - jnp.max(z, axis=1, keepdims=True)
    ez = jnp.exp(z)
    p_ref[...] = ez / jnp.sum(ez, axis=1, keepdims=True)


def _router(x2d, ln_g, ln_b, Wr):
    return pl.pallas_call(
        _router_body,
        grid=(T // TB,),
        in_specs=[
            pl.BlockSpec((TB, D), lambda t: (t, 0)),
            pl.BlockSpec((1, D), lambda t: (0, 0)),
            pl.BlockSpec((1, D), lambda t: (0, 0)),
            pl.BlockSpec((E, D), lambda t: (0, 0)),
        ],
        out_specs=[
            pl.BlockSpec((TB, D), lambda t: (t, 0)),
            pl.BlockSpec((TB, E), lambda t: (t, 0)),
        ],
        out_shape=[
            jax.ShapeDtypeStruct((T, D), jnp.bfloat16),
            jax.ShapeDtypeStruct((T, E), jnp.float32),
        ],
    )(x2d, ln_g.reshape(1, D), ln_b.reshape(1, D), Wr)


def _dispatch_metadata(probs):
    w, idx = lax.top_k(probs, K)
    flat_e = idx.reshape(-1)

    onehot = (flat_e[:, None] == jnp.arange(E)[None, :]).astype(jnp.int32)
    csum = jnp.cumsum(onehot, axis=0)
    counts = csum[-1]
    rank = jnp.take_along_axis(csum - onehot, flat_e[:, None], axis=1)[:, 0]

    tiles_e = (counts + M - 1) // M
    tile_start = jnp.concatenate([jnp.zeros((1,), jnp.int32),
                                  jnp.cumsum(tiles_e).astype(jnp.int32)])
    pad_start = tile_start[:E] * M

    row_of_flat = (pad_start[flat_e] + rank).astype(jnp.int32)
    r01 = row_of_flat.reshape(T, K)

    n_act = tile_start[E]
    raw_tile_e = jnp.searchsorted(
        tile_start[1:], jnp.arange(NT, dtype=jnp.int32), side="right"
    ).astype(jnp.int32)
    active = (jnp.arange(NT, dtype=jnp.int32) < n_act).astype(jnp.int32)
    last_e = raw_tile_e[jnp.maximum(n_act - 1, 0)]
    tile_e = jnp.where(active == 1, jnp.minimum(raw_tile_e, E - 1), last_e)
    blk = jnp.where(active == 1, jnp.arange(NT, dtype=jnp.int32),
                    jnp.maximum(n_act - 1, 0)).astype(jnp.int32)

    wh = w.astype(jnp.bfloat16)
    wl = (w - wh.astype(jnp.float32)).astype(jnp.bfloat16)
    w0cat = jnp.stack([wh[:, 0], wl[:, 0]], axis=1).reshape(1, T, 2)
    w1cat = jnp.stack([wh[:, 1], wl[:, 1]], axis=1).reshape(1, T, 2)
    return (w0cat, w1cat, r01[:, 0], r01[:, 1], tile_e, active, blk)


def _mm_body(te_ref, act_ref, blk_ref, xnb_ref, r0_ref, r1_ref, w0_ref,
             w1c_ref, wm1_ref, wm2_ref, ys_ref, xg_ref, wrow_ref):
    t = pl.program_id(0)
    f = pl.program_id(1)
    is_act = act_ref[t] == 1

    @pl.when(jnp.logical_and(is_act, f == 0))
    def _():
        ys_ref[...] = jnp.zeros_like(ys_ref)
        rid = lax.broadcasted_iota(jnp.int32, (M, T), 0) + t * M
        oh0 = (rid == r0_ref[0]).astype(jnp.bfloat16)
        oh1 = (rid == r1_ref[0]).astype(jnp.bfloat16)
        xg = lax.dot_general(oh0 + oh1, xnb_ref[...], (((1,), (0,)), ((), ())),
                             preferred_element_type=jnp.float32)
        xg_ref[...] = xg
        a = lax.dot_general(oh0, w0_ref[0], (((1,), (0,)), ((), ())),
                            preferred_element_type=jnp.float32)
        b = lax.dot_general(oh1, w1c_ref[0], (((1,), (0,)), ((), ())),
                            preferred_element_type=jnp.float32)
        wrow_ref[...] = jnp.sum(a + b, axis=1, keepdims=True)

    @pl.when(is_act)
    def _():
        x = xg_ref[...]
        h = lax.dot_general(x, wm1_ref[0], (((1,), (1,)), ((), ())),
                            preferred_element_type=jnp.float32)
        h = h * lax.logistic(h)
        y = lax.dot_general(h, wm2_ref[0], (((1,), (1,)), ((), ())),
                            preferred_element_type=jnp.float32)
        ys_ref[...] += y

    @pl.when(jnp.logical_and(is_act, f == FC - 1))
    def _():
        ys_ref[...] *= wrow_ref[...]


def _grouped_ffn(xnb, r0, r1, w0cat, w1cat, W1, W2, tile_e, active, blk):
    grid_spec = pltpu.PrefetchScalarGridSpec(
        num_scalar_prefetch=3,
        grid=(NT, FC),
        in_specs=[
            pl.BlockSpec((T, D), lambda t, f, te, act, blk: (0, 0)),
            pl.BlockSpec((1, 1, T), lambda t, f, te, act, blk: (0, 0, 0)),
            pl.BlockSpec((1, 1, T), lambda t, f, te, act, blk: (0, 0, 0)),
            pl.BlockSpec((1, T, 2), lambda t, f, te, act, blk: (0, 0, 0)),
            pl.BlockSpec((1, T, 2), lambda t, f, te, act, blk: (0, 0, 0)),
            pl.BlockSpec(
                (1, F, D),
                lambda t, f, te, act, blk:
                    (te[t], jnp.where(act[t] == 1, f, FC - 1), 0)),
            pl.BlockSpec(
                (1, D, F),
                lambda t, f, te, act, blk:
                    (te[t], 0, jnp.where(act[t] == 1, f, FC - 1))),
        ],
        out_specs=pl.BlockSpec((M, D), lambda t, f, te, act, blk: (blk[t], 0)),
        scratch_shapes=[pltpu.VMEM((M, D), jnp.float32),
                        pltpu.VMEM((M, 1), jnp.float32)],
    )
    return pl.pallas_call(
        _mm_body,
        grid_spec=grid_spec,
        out_shape=jax.ShapeDtypeStruct((R, D), jnp.float32),
        compiler_params=pltpu.CompilerParams(
            dimension_semantics=("arbitrary", "arbitrary")),
    )(tile_e, active, blk, xnb, r0.reshape(1, 1, T), r1.reshape(1, 1, T),
      w0cat, w1cat, W1, W2)


CCH = 16


def _combine_body(ys_hbm, r0_hbm, r1_hbm, x_hbm, out_hbm,
                  i0_v, i1_v, a_v, b_v, c_v, sem0, sem1):
    wid = lax.axis_index("s") * NC + lax.axis_index("c")
    per_w = T // NW

    def chunk(i, _):
        base = wid * per_w + i * CCH
        pltpu.sync_copy(r0_hbm.at[pl.ds(base, CCH)], i0_v)
        pltpu.sync_copy(r1_hbm.at[pl.ds(base, CCH)], i1_v)
        cp0 = pltpu.async_copy(ys_hbm.at[i0_v], a_v, sem0)
        cp1 = pltpu.async_copy(ys_hbm.at[i1_v], b_v, sem1)
        pltpu.sync_copy(x_hbm.at[pl.ds(base, CCH)], c_v)
        cp0.wait()
        cp1.wait()

        def row(r, _):
            for j in range(D // LANES):
                s = pl.ds(j * LANES, LANES)
                c_v[r, s] = c_v[r, s] + a_v[r, s] + b_v[r, s]
            return _

        lax.fori_loop(0, CCH, row, None)
        pltpu.sync_copy(c_v, out_hbm.at[pl.ds(base, CCH)])
        return _

    lax.fori_loop(0, per_w // CCH, chunk, None)


def _sc_combine(ys, r0, r1, x2d):
    k = functools.partial(
        pl.kernel,
        out_type=jax.ShapeDtypeStruct((T, D), jnp.float32),
        mesh=_sc_mesh(),
        scratch_types=[
            pltpu.VMEM((CCH,), jnp.int32),
            pltpu.VMEM((CCH,), jnp.int32),
            pltpu.VMEM((CCH, D), jnp.float32),
            pltpu.VMEM((CCH, D), jnp.float32),
            pltpu.VMEM((CCH, D), jnp.float32),
            pltpu.SemaphoreType.DMA,
            pltpu.SemaphoreType.DMA,
        ],
    )(_combine_body)
    return k(ys, r0, r1, x2d)


def kernel(x, ln_g, ln_b, Wr, W1, W2):
    x2d = x.reshape(T, D)
    xnb, probs = _router(x2d, ln_g, ln_b, Wr)
    w0cat, w1cat, r0, r1, tile_e, active, blk = _dispatch_metadata(probs)
    ys = _grouped_ffn(xnb, r0, r1, w0cat, w1cat, W1, W2, tile_e, active, blk)
    out = _sc_combine(ys, r0, r1, x2d)
    return out.reshape(1, T, D)
